# baseline (speedup 1.0000x reference)
import functools
import math

import jax
import jax.numpy as jnp
import numpy as np
from jax import lax
from jax.experimental import pallas as pl
from jax.experimental.pallas import tpu as pltpu
from jax.experimental.pallas import tpu_sc as plsc

_D = 1024
_SEQ = 4096
_BATCH = 4
_B = _BATCH * _SEQ
_BH = _B // 2
_NW = 32
_BPW = _BH // _NW
_R = 16
_CH = _BPW // _R
_NSLOT = 4
_SCALE = math.sqrt(_D)
_BR = 2048
_PB = _SEQ // _BR


def _pe_np(seq_len: int, d_model: int) -> np.ndarray:
    pos = np.arange(seq_len, dtype=np.float32)[:, None]
    div = np.exp(
        np.arange(0, d_model, 2, dtype=np.float32) * (-math.log(10000.0) / d_model)
    )
    pe = np.zeros((seq_len, d_model), dtype=np.float32)
    pe[:, 0::2] = np.sin(pos * div)
    pe[:, 1::2] = np.cos(pos * div)
    return pe


_PE = _pe_np(_SEQ, _D)

_mesh = plsc.VectorSubcoreMesh(core_axis_name="c", subcore_axis_name="s")


@functools.partial(
    pl.kernel,
    out_type=jax.ShapeDtypeStruct((_BH, _D), jnp.float32),
    mesh=_mesh,
    scratch_types=[
        pltpu.VMEM((_BPW,), jnp.int32),
        pltpu.VMEM((_NSLOT, _R, _D), jnp.float32),
        pltpu.SemaphoreType.DMA,
        pltpu.SemaphoreType.DMA,
        pltpu.SemaphoreType.DMA,
        pltpu.SemaphoreType.DMA,
        pltpu.SemaphoreType.DMA,
        pltpu.SemaphoreType.DMA,
        pltpu.SemaphoreType.DMA,
        pltpu.SemaphoreType.DMA,
    ],
)
def _gather_half(idx_hbm, table_hbm, out_hbm, idx_v, rows_v,
                 isem0, isem1, isem2, isem3, osem0, osem1, osem2, osem3):
    c = lax.axis_index("c")
    s = lax.axis_index("s")
    wid = s * 2 + c
    base = wid * _BPW
    in_sems = (isem0, isem1, isem2, isem3)
    out_sems = (osem0, osem1, osem2, osem3)

    pltpu.sync_copy(idx_hbm.at[pl.ds(base, _BPW)], idx_v)

    def issue_in(g, slot):
        iv = idx_v.at[pl.ds(g * _R, _R)]
        pltpu.async_copy(table_hbm.at[iv], rows_v.at[slot], in_sems[slot])

    for g0 in range(_NSLOT - 1):
        issue_in(g0, g0)

    def body(q, carry):
        for slot in range(_NSLOT):
            g = q * _NSLOT + slot
            iv = idx_v.at[pl.ds(g * _R, _R)]
            pltpu.make_async_copy(table_hbm.at[iv], rows_v.at[slot],
                                  in_sems[slot]).wait()
            pltpu.async_copy(rows_v.at[slot],
                             out_hbm.at[pl.ds(base + g * _R, _R)],
                             out_sems[slot])

            prev_slot = (slot + _NSLOT - 1) % _NSLOT
            @pl.when(g >= 1)
            def _():
                pltpu.make_async_copy(rows_v.at[prev_slot],
                                      out_hbm.at[pl.ds(0, _R)],
                                      out_sems[prev_slot]).wait()

            @pl.when(g + _NSLOT - 1 < _CH)
            def _():
                issue_in(g + _NSLOT - 1, prev_slot)
        return carry

    lax.fori_loop(0, _CH // _NSLOT, body, 0)

    last_slot = (_CH - 1) % _NSLOT
    pltpu.make_async_copy(rows_v.at[last_slot], out_hbm.at[pl.ds(0, _R)],
                          out_sems[last_slot]).wait()


def _fuse0_body(tmp_ref, pe_ref, out_ref):
    out_ref[...] = tmp_ref[...] * _SCALE + pe_ref[...]


def _fuse1_body(tmp_ref, pe_ref, prev_ref, out_ref):
    del prev_ref
    out_ref[...] = tmp_ref[...] * _SCALE + pe_ref[...]


def _make_fuse(h, body, n_in, br):
    pb = _SEQ // br
    out_map = lambda i, bl: ((2 * h + bl) * pb + i, 0)
    in_specs = [
        pl.BlockSpec((br, _D), lambda i, bl: (bl * pb + i, 0)),
        pl.BlockSpec((br, _D), lambda i, bl: (i, 0)),
    ]
    kwargs = {}
    if n_in == 3:
        in_specs.append(pl.BlockSpec((br, _D), out_map))
        kwargs["input_output_aliases"] = {2: 0}
    return pl.pallas_call(
        body,
        out_shape=jax.ShapeDtypeStruct((_B, _D), jnp.float32),
        grid=(pb, 2),
        in_specs=in_specs,
        out_specs=pl.BlockSpec((br, _D), out_map),
        **kwargs,
    )


_fuse0 = _make_fuse(0, _fuse0_body, 2, 2048)
_fuse1 = _make_fuse(1, _fuse1_body, 3, 1024)


def kernel(x, table):
    idx = x.reshape(_B)
    pe = jnp.asarray(_PE)
    tmp0 = _gather_half(idx[:_BH], table)
    tmp1 = _gather_half(idx[_BH:], table)
    out0 = _fuse0(tmp0, pe)
    out = _fuse1(tmp1, pe, out0)
    return out.reshape(_BATCH, _SEQ, _D)


# speedup vs baseline: 1.1299x; 1.1299x over previous
import functools
import math

import jax
import jax.numpy as jnp
import numpy as np
from jax import lax
from jax.experimental import pallas as pl
from jax.experimental.pallas import tpu as pltpu
from jax.experimental.pallas import tpu_sc as plsc

_D = 1024
_SEQ = 4096
_BATCH = 4
_B = _BATCH * _SEQ
_NW = 32
_BPW = _B // _NW
_R = 16
_CH = _BPW // _R
_NSLOT = 4
_SCALE = math.sqrt(_D)
_BR = 2048
_PB = _SEQ // _BR


def _pe_np(seq_len: int, d_model: int) -> np.ndarray:
    pos = np.arange(seq_len, dtype=np.float32)[:, None]
    div = np.exp(
        np.arange(0, d_model, 2, dtype=np.float32) * (-math.log(10000.0) / d_model)
    )
    pe = np.zeros((seq_len, d_model), dtype=np.float32)
    pe[:, 0::2] = np.sin(pos * div)
    pe[:, 1::2] = np.cos(pos * div)
    return pe


_PE = _pe_np(_SEQ, _D)

_mesh = plsc.VectorSubcoreMesh(core_axis_name="c", subcore_axis_name="s")


@functools.partial(
    pl.kernel,
    out_type=jax.ShapeDtypeStruct((_B, _D), jnp.float32),
    mesh=_mesh,
    scratch_types=[
        pltpu.VMEM((_BPW,), jnp.int32),
        pltpu.VMEM((_NSLOT, _R, _D), jnp.float32),
        pltpu.SemaphoreType.DMA,
        pltpu.SemaphoreType.DMA,
        pltpu.SemaphoreType.DMA,
        pltpu.SemaphoreType.DMA,
        pltpu.SemaphoreType.DMA,
        pltpu.SemaphoreType.DMA,
        pltpu.SemaphoreType.DMA,
        pltpu.SemaphoreType.DMA,
    ],
)
def _gather(idx_hbm, table_hbm, out_hbm, idx_v, rows_v,
            isem0, isem1, isem2, isem3, osem0, osem1, osem2, osem3):
    c = lax.axis_index("c")
    s = lax.axis_index("s")
    wid = s * 2 + c
    base = wid * _BPW
    in_sems = (isem0, isem1, isem2, isem3)
    out_sems = (osem0, osem1, osem2, osem3)

    pltpu.sync_copy(idx_hbm.at[wid >> 3, pl.ds((wid & 7) * _BPW, _BPW)],
                    idx_v)

    def issue_in(g, slot):
        iv = idx_v.at[pl.ds(g * _R, _R)]
        pltpu.async_copy(table_hbm.at[iv], rows_v.at[slot], in_sems[slot])

    for g0 in range(_NSLOT - 1):
        issue_in(g0, g0)

    def body(q, carry):
        for slot in range(_NSLOT):
            g = q * _NSLOT + slot
            iv = idx_v.at[pl.ds(g * _R, _R)]
            pltpu.make_async_copy(table_hbm.at[iv], rows_v.at[slot],
                                  in_sems[slot]).wait()
            pltpu.async_copy(rows_v.at[slot],
                             out_hbm.at[pl.ds(base + g * _R, _R)],
                             out_sems[slot])

            prev_slot = (slot + _NSLOT - 1) % _NSLOT
            @pl.when(g >= 1)
            def _():
                pltpu.make_async_copy(rows_v.at[prev_slot],
                                      out_hbm.at[pl.ds(0, _R)],
                                      out_sems[prev_slot]).wait()

            @pl.when(g + _NSLOT - 1 < _CH)
            def _():
                issue_in(g + _NSLOT - 1, prev_slot)
        return carry

    lax.fori_loop(0, _CH // _NSLOT, body, 0)

    last_slot = (_CH - 1) % _NSLOT
    pltpu.make_async_copy(rows_v.at[last_slot], out_hbm.at[pl.ds(0, _R)],
                          out_sems[last_slot]).wait()


def _fuse_body(tmp_ref, pe_ref, out_ref):
    out_ref[...] = tmp_ref[...] * _SCALE + pe_ref[...]


_fuse = pl.pallas_call(
    _fuse_body,
    out_shape=jax.ShapeDtypeStruct((_B, _D), jnp.float32),
    grid=(_PB, _BATCH),
    in_specs=[
        pl.BlockSpec((_BR, _D), lambda i, b: (b * _PB + i, 0)),
        pl.BlockSpec((_BR, _D), lambda i, b: (i, 0)),
    ],
    out_specs=pl.BlockSpec((_BR, _D), lambda i, b: (b * _PB + i, 0)),
)


def kernel(x, table):
    tmp = _gather(x, table)
    pe = jnp.asarray(_PE)
    out = _fuse(tmp, pe)
    return out.reshape(_BATCH, _SEQ, _D)


# speedup vs baseline: 1.1579x; 1.0248x over previous
import functools
import math

import jax
import jax.numpy as jnp
import numpy as np
from jax import lax
from jax.experimental import pallas as pl
from jax.experimental.pallas import tpu as pltpu
from jax.experimental.pallas import tpu_sc as plsc

_D = 1024
_SEQ = 4096
_BATCH = 4
_B = _BATCH * _SEQ
_NW = 32
_BPW = _B // _NW
_R = 16
_CH = _BPW // _R
_NSLOT = 4
_SCALE = math.sqrt(_D)
_BR = 2048
_PB = _SEQ // _BR


def _pe_np(seq_len: int, d_model: int) -> np.ndarray:
    pos = np.arange(seq_len, dtype=np.float32)[:, None]
    div = np.exp(
        np.arange(0, d_model, 2, dtype=np.float32) * (-math.log(10000.0) / d_model)
    )
    pe = np.zeros((seq_len, d_model), dtype=np.float32)
    pe[:, 0::2] = np.sin(pos * div)
    pe[:, 1::2] = np.cos(pos * div)
    return pe


_PE = _pe_np(_SEQ, _D)
_PE_BF16 = _PE.astype(jnp.bfloat16)

_mesh = plsc.VectorSubcoreMesh(core_axis_name="c", subcore_axis_name="s")


@functools.partial(
    pl.kernel,
    out_type=jax.ShapeDtypeStruct((_B, _D), jnp.float32),
    mesh=_mesh,
    scratch_types=[
        pltpu.VMEM((_BPW,), jnp.int32),
        pltpu.VMEM((_NSLOT, _R, _D), jnp.float32),
        pltpu.SemaphoreType.DMA,
        pltpu.SemaphoreType.DMA,
        pltpu.SemaphoreType.DMA,
        pltpu.SemaphoreType.DMA,
        pltpu.SemaphoreType.DMA,
        pltpu.SemaphoreType.DMA,
        pltpu.SemaphoreType.DMA,
        pltpu.SemaphoreType.DMA,
    ],
)
def _gather(idx_hbm, table_hbm, out_hbm, idx_v, rows_v,
            isem0, isem1, isem2, isem3, osem0, osem1, osem2, osem3):
    c = lax.axis_index("c")
    s = lax.axis_index("s")
    wid = s * 2 + c
    base = wid * _BPW
    in_sems = (isem0, isem1, isem2, isem3)
    out_sems = (osem0, osem1, osem2, osem3)

    pltpu.sync_copy(idx_hbm.at[wid >> 3, pl.ds((wid & 7) * _BPW, _BPW)],
                    idx_v)

    def issue_in(g, slot):
        iv = idx_v.at[pl.ds(g * _R, _R)]
        pltpu.async_copy(table_hbm.at[iv], rows_v.at[slot], in_sems[slot])

    for g0 in range(_NSLOT - 1):
        issue_in(g0, g0)

    def body(q, carry):
        for slot in range(_NSLOT):
            g = q * _NSLOT + slot
            iv = idx_v.at[pl.ds(g * _R, _R)]
            pltpu.make_async_copy(table_hbm.at[iv], rows_v.at[slot],
                                  in_sems[slot]).wait()
            pltpu.async_copy(rows_v.at[slot],
                             out_hbm.at[pl.ds(base + g * _R, _R)],
                             out_sems[slot])

            prev_slot = (slot + _NSLOT - 1) % _NSLOT
            @pl.when(g >= 1)
            def _():
                pltpu.make_async_copy(rows_v.at[prev_slot],
                                      out_hbm.at[pl.ds(0, _R)],
                                      out_sems[prev_slot]).wait()

            @pl.when(g + _NSLOT - 1 < _CH)
            def _():
                issue_in(g + _NSLOT - 1, prev_slot)
        return carry

    lax.fori_loop(0, _CH // _NSLOT, body, 0)

    last_slot = (_CH - 1) % _NSLOT
    pltpu.make_async_copy(rows_v.at[last_slot], out_hbm.at[pl.ds(0, _R)],
                          out_sems[last_slot]).wait()


def _fuse_body(tmp_ref, pe_ref, out_ref):
    out_ref[...] = tmp_ref[...] * _SCALE + pe_ref[...].astype(jnp.float32)


_fuse = pl.pallas_call(
    _fuse_body,
    out_shape=jax.ShapeDtypeStruct((_B, _D), jnp.float32),
    grid=(_PB, _BATCH),
    in_specs=[
        pl.BlockSpec((_BR, _D), lambda i, b: (b * _PB + i, 0)),
        pl.BlockSpec((_BR, _D), lambda i, b: (i, 0)),
    ],
    out_specs=pl.BlockSpec((_BR, _D), lambda i, b: (b * _PB + i, 0)),
)


def kernel(x, table):
    tmp = _gather(x, table)
    pe = jnp.asarray(_PE_BF16)
    out = _fuse(tmp, pe)
    return out.reshape(_BATCH, _SEQ, _D)
